# baseline (speedup 1.0000x reference)
import functools

import jax
import jax.numpy as jnp
from jax import lax
from jax.experimental import pallas as pl
from jax.experimental.pallas import tpu as pltpu
from jax.experimental.pallas import tpu_sc as plsc

BATCH = 16384
HIST_LEN = 50
EMBED_DIM = 32

SUB = 128
SG = BATCH // SUB
UNITS = HIST_LEN * SG
NC, NS = 2, 16
NW = NC * NS
UPW = UNITS // NW
DG = EMBED_DIM // 8

_MESH = plsc.VectorSubcoreMesh(core_axis_name="c", subcore_axis_name="s")

_TC_COLS = 2048
_TC_GRID = 123
_OUT_ROWS = _TC_GRID * _TC_COLS


def _tc_transpose_body(x0_ref, x1_ref, x2_ref, x3_ref, o_ref):
    row = lax.broadcasted_iota(jnp.int32, (EMBED_DIM, EMBED_DIM), 0)
    col = lax.broadcasted_iota(jnp.int32, (EMBED_DIM, EMBED_DIM), 1)
    eye = (row == col).astype(jnp.float32)
    parts = [
        lax.dot_general(x_ref[...], eye, (((0,), (0,)), ((), ())),
                        precision=lax.Precision.HIGHEST)
        for x_ref in (x0_ref, x1_ref, x2_ref, x3_ref)
    ]
    o_ref[...] = jnp.concatenate(parts, axis=1)


def _tc_transpose(table_t):
    last_block = (1000000 - 1) // _TC_COLS
    specs = [
        pl.BlockSpec(
            (EMBED_DIM, _TC_COLS),
            functools.partial(
                lambda a, j: (0, jnp.minimum(4 * j + a, last_block)), a))
        for a in range(4)
    ]
    return pl.pallas_call(
        _tc_transpose_body,
        grid=(_TC_GRID,),
        in_specs=specs,
        out_specs=pl.BlockSpec((_TC_COLS, 128), lambda j: (j, 0)),
        out_shape=jax.ShapeDtypeStruct((_OUT_ROWS, 128), jnp.float32),
    )(table_t, table_t, table_t, table_t)


@functools.partial(
    pl.kernel,
    mesh=_MESH,
    out_type=jax.ShapeDtypeStruct((HIST_LEN, DG, SG, 8, SUB), jnp.float32),
    compiler_params=pltpu.CompilerParams(
        use_tc_tiling_on_sc=False, needs_layout_passes=False),
    scratch_types=[
        pltpu.VMEM((UPW, SUB), jnp.int32),
        pltpu.VMEM((4, SUB, EMBED_DIM), jnp.float32),
        pltpu.VMEM((2, EMBED_DIM, SUB), jnp.float32),
        pltpu.SemaphoreType.DMA((4,)),
        pltpu.SemaphoreType.DMA((2,)),
    ],
)
def _gather_sc(table_hbm, idx_hbm, out_hbm, idx_all, rows_g, rows_t, sem_g, sem_o):
    wid = lax.axis_index("s") * NC + lax.axis_index("c")
    u0 = wid * UPW

    pltpu.sync_copy(idx_hbm.at[pl.ds(u0, UPW)], idx_all)

    def gather(t):
        q = lax.rem(t, 4)
        return pltpu.make_async_copy(
            table_hbm.at[idx_all.at[t]], rows_g.at[q], sem_g.at[q])

    def store(t, dg):
        u = u0 + t
        h = lax.div(u, SG)
        sg = lax.rem(u, SG)
        q = lax.rem(t, 2)
        return pltpu.make_async_copy(
            rows_t.at[q, pl.ds(dg * 8, 8)], out_hbm.at[h, dg, sg], sem_o.at[q])

    lanevec = lax.iota(jnp.int32, 16)
    riota = [lanevec + 16 * k for k in range(8)]

    gather(0).start()
    gather(1).start()
    gather(2).start()

    def unit(t, carry):
        q4 = lax.rem(t, 4)
        q2 = lax.rem(t, 2)

        @pl.when(t >= 2)
        def _():
            for dg in range(DG):
                store(t - 2, dg).wait()

        gather(t).wait()

        @pl.when(t + 3 < UPW)
        def _():
            gather(t + 3).start()

        src = rows_g.at[q4]
        qvec = jnp.full((16,), 0, jnp.int32) + q2

        for d0 in range(EMBED_DIM):
            cvec = (lanevec + d0) & (EMBED_DIM - 1)
            for k in range(8):
                v = plsc.load_gather(src, [riota[k], cvec])
                plsc.store_scatter(rows_t, [qvec, cvec, riota[k]], v)

        for dg in range(DG):
            store(t, dg).start()
        return carry

    lax.fori_loop(0, UPW, unit, 0)

    for dg in range(DG):
        store(UPW - 2, dg).wait()
        store(UPW - 1, dg).wait()


def kernel(nouns_idx_tensor, histwords_embeddings):
    idx = nouns_idx_tensor.astype(jnp.int32).T.reshape(UNITS, SUB)
    idx = ((idx >> 13) << 13) + ((idx & 2047) << 2) + ((idx >> 11) & 3)
    table_l = _tc_transpose(histwords_embeddings.T)
    out5 = _gather_sc(table_l.reshape(4 * _OUT_ROWS, EMBED_DIM), idx)
    return out5.transpose(2, 4, 0, 1, 3).reshape(BATCH, HIST_LEN, EMBED_DIM)


# speedup vs baseline: 1.4755x; 1.4755x over previous
import functools

import jax
import jax.numpy as jnp
from jax import lax
from jax.experimental import pallas as pl
from jax.experimental.pallas import tpu as pltpu
from jax.experimental.pallas import tpu_sc as plsc

BATCH = 16384
HIST_LEN = 50
EMBED_DIM = 32

SUB = 128
SG = BATCH // SUB
UNITS = HIST_LEN * SG
NC, NS = 2, 16
NW = NC * NS
UPW = UNITS // NW
DG = EMBED_DIM // 8

_MESH = plsc.VectorSubcoreMesh(core_axis_name="c", subcore_axis_name="s")

_TC_COLS = 2048
_TC_GRID = 123
_OUT_ROWS = _TC_GRID * _TC_COLS


def _tc_transpose_body(x0_ref, x1_ref, x2_ref, x3_ref, o_ref):
    parts = [x_ref[...].T for x_ref in (x0_ref, x1_ref, x2_ref, x3_ref)]
    o_ref[...] = jnp.concatenate(parts, axis=1)


def _tc_transpose(table_t):
    last_block = (1000000 - 1) // _TC_COLS
    specs = [
        pl.BlockSpec(
            (EMBED_DIM, _TC_COLS),
            functools.partial(
                lambda a, j: (0, jnp.minimum(4 * j + a, last_block)), a))
        for a in range(4)
    ]
    return pl.pallas_call(
        _tc_transpose_body,
        grid=(_TC_GRID,),
        in_specs=specs,
        out_specs=pl.BlockSpec((_TC_COLS, 128), lambda j: (j, 0)),
        out_shape=jax.ShapeDtypeStruct((_OUT_ROWS, 128), jnp.float32),
    )(table_t, table_t, table_t, table_t)


@functools.partial(
    pl.kernel,
    mesh=_MESH,
    out_type=jax.ShapeDtypeStruct((HIST_LEN, DG, SG, 8, SUB), jnp.float32),
    compiler_params=pltpu.CompilerParams(
        use_tc_tiling_on_sc=False, needs_layout_passes=False),
    scratch_types=[
        pltpu.VMEM((UPW, SUB), jnp.int32),
        pltpu.VMEM((4, SUB, EMBED_DIM), jnp.float32),
        pltpu.VMEM((2, EMBED_DIM, SUB), jnp.float32),
        pltpu.SemaphoreType.DMA((4,)),
        pltpu.SemaphoreType.DMA((2,)),
    ],
)
def _gather_sc(table_hbm, idx_hbm, out_hbm, idx_all, rows_g, rows_t, sem_g, sem_o):
    wid = lax.axis_index("s") * NC + lax.axis_index("c")
    u0 = wid * UPW

    pltpu.sync_copy(idx_hbm.at[pl.ds(u0, UPW)], idx_all)

    def gather(t):
        q = lax.rem(t, 4)
        return pltpu.make_async_copy(
            table_hbm.at[idx_all.at[t]], rows_g.at[q], sem_g.at[q])

    def store(t, dg):
        u = u0 + t
        h = lax.div(u, SG)
        sg = lax.rem(u, SG)
        q = lax.rem(t, 2)
        return pltpu.make_async_copy(
            rows_t.at[q, pl.ds(dg * 8, 8)], out_hbm.at[h, dg, sg], sem_o.at[q])

    lanevec = lax.iota(jnp.int32, 16)
    riota = [lanevec + 16 * k for k in range(8)]
    cvecs = [(lanevec + d0) & (EMBED_DIM - 1) for d0 in range(EMBED_DIM)]

    gather(0).start()
    gather(1).start()
    gather(2).start()

    def unit(t, carry):
        q4 = lax.rem(t, 4)
        q2 = lax.rem(t, 2)

        @pl.when(t >= 2)
        def _():
            for dg in range(DG):
                store(t - 2, dg).wait()

        gather(t).wait()

        @pl.when(t + 3 < UPW)
        def _():
            gather(t + 3).start()

        src = rows_g.at[q4]
        qvec = jnp.full((16,), 0, jnp.int32) + q2

        for d0 in range(EMBED_DIM):
            cvec = cvecs[d0]
            for k in range(8):
                v = plsc.load_gather(src, [riota[k], cvec])
                plsc.store_scatter(rows_t, [qvec, cvec, riota[k]], v)

        for dg in range(DG):
            store(t, dg).start()
        return carry

    lax.fori_loop(0, UPW, unit, 0)

    for dg in range(DG):
        store(UPW - 2, dg).wait()
        store(UPW - 1, dg).wait()


def kernel(nouns_idx_tensor, histwords_embeddings):
    idx = nouns_idx_tensor.astype(jnp.int32).T.reshape(UNITS, SUB)
    idx = ((idx >> 13) << 13) + ((idx & 2047) << 2) + ((idx >> 11) & 3)
    table_l = _tc_transpose(histwords_embeddings.T)
    out5 = _gather_sc(table_l.reshape(4 * _OUT_ROWS, EMBED_DIM), idx)
    return out5.transpose(2, 4, 0, 1, 3).reshape(BATCH, HIST_LEN, EMBED_DIM)


# speedup vs baseline: 1.6369x; 1.1094x over previous
import functools

import jax
import jax.numpy as jnp
from jax import lax
from jax.experimental import pallas as pl
from jax.experimental.pallas import tpu as pltpu
from jax.experimental.pallas import tpu_sc as plsc

BATCH = 16384
HIST_LEN = 50
EMBED_DIM = 32

SUB = 128
SG = BATCH // SUB
UNITS = HIST_LEN * SG
NC, NS = 2, 16
NW = NC * NS
UPW = UNITS // NW
DG = EMBED_DIM // 8

_MESH = plsc.VectorSubcoreMesh(core_axis_name="c", subcore_axis_name="s")

_TC_COLS = 4096
_TC_GRID = 62
_OUT_ROWS = _TC_GRID * _TC_COLS


def _tc_transpose_body(x0_ref, x1_ref, x2_ref, x3_ref, o_ref):
    parts = [x_ref[...].T for x_ref in (x0_ref, x1_ref, x2_ref, x3_ref)]
    o_ref[...] = jnp.concatenate(parts, axis=1)


def _tc_transpose(table_t):
    last_block = (1000000 - 1) // _TC_COLS
    specs = [
        pl.BlockSpec(
            (EMBED_DIM, _TC_COLS),
            functools.partial(
                lambda a, j: (0, jnp.minimum(4 * j + a, last_block)), a))
        for a in range(4)
    ]
    return pl.pallas_call(
        _tc_transpose_body,
        grid=(_TC_GRID,),
        in_specs=specs,
        out_specs=pl.BlockSpec((_TC_COLS, 128), lambda j: (j, 0)),
        out_shape=jax.ShapeDtypeStruct((_OUT_ROWS, 128), jnp.float32),
    )(table_t, table_t, table_t, table_t)


@functools.partial(
    pl.kernel,
    mesh=_MESH,
    out_type=jax.ShapeDtypeStruct((HIST_LEN, DG, SG, 8, SUB), jnp.float32),
    compiler_params=pltpu.CompilerParams(
        use_tc_tiling_on_sc=False, needs_layout_passes=False),
    scratch_types=[
        pltpu.VMEM((UPW, SUB), jnp.int32),
        pltpu.VMEM((4, SUB, EMBED_DIM), jnp.float32),
        pltpu.VMEM((2, EMBED_DIM, SUB), jnp.float32),
        pltpu.SemaphoreType.DMA((4,)),
        pltpu.SemaphoreType.DMA((2,)),
    ],
)
def _gather_sc(table_hbm, idx_hbm, out_hbm, idx_all, rows_g, rows_t, sem_g, sem_o):
    wid = lax.axis_index("s") * NC + lax.axis_index("c")
    u0 = wid * UPW

    pltpu.sync_copy(idx_hbm.at[pl.ds(u0, UPW)], idx_all)

    def gather(t):
        q = lax.rem(t, 4)
        return pltpu.make_async_copy(
            table_hbm.at[idx_all.at[t]], rows_g.at[q], sem_g.at[q])

    def store(t, dg):
        u = u0 + t
        h = lax.div(u, SG)
        sg = lax.rem(u, SG)
        q = lax.rem(t, 2)
        return pltpu.make_async_copy(
            rows_t.at[q, pl.ds(dg * 8, 8)], out_hbm.at[h, dg, sg], sem_o.at[q])

    lanevec = lax.iota(jnp.int32, 16)
    riota = [lanevec + 16 * k for k in range(8)]
    cvecs = [(lanevec + d0) & (EMBED_DIM - 1) for d0 in range(EMBED_DIM)]

    gather(0).start()
    gather(1).start()
    gather(2).start()

    def unit(t, carry):
        q4 = lax.rem(t, 4)
        q2 = lax.rem(t, 2)

        @pl.when(t >= 2)
        def _():
            for dg in range(DG):
                store(t - 2, dg).wait()

        gather(t).wait()

        @pl.when(t + 3 < UPW)
        def _():
            gather(t + 3).start()

        src = rows_g.at[q4]
        qvec = jnp.full((16,), 0, jnp.int32) + q2

        for d0 in range(EMBED_DIM):
            cvec = cvecs[d0]
            vs = [plsc.load_gather(src, [riota[k], cvec]) for k in range(8)]
            for k in range(8):
                plsc.store_scatter(rows_t, [qvec, cvec, riota[k]], vs[k])

        for dg in range(DG):
            store(t, dg).start()
        return carry

    lax.fori_loop(0, UPW, unit, 0)

    for dg in range(DG):
        store(UPW - 2, dg).wait()
        store(UPW - 1, dg).wait()


def kernel(nouns_idx_tensor, histwords_embeddings):
    idx = nouns_idx_tensor.astype(jnp.int32).T.reshape(UNITS, SUB)
    idx = ((idx >> 14) << 14) + ((idx & 4095) << 2) + ((idx >> 12) & 3)
    table_l = _tc_transpose(histwords_embeddings.T)
    out5 = _gather_sc(table_l.reshape(4 * _OUT_ROWS, EMBED_DIM), idx)
    return out5.transpose(2, 4, 0, 1, 3).reshape(BATCH, HIST_LEN, EMBED_DIM)


# speedup vs baseline: 2.2399x; 1.3684x over previous
import functools

import jax
import jax.numpy as jnp
from jax import lax
from jax.experimental import pallas as pl
from jax.experimental.pallas import tpu as pltpu
from jax.experimental.pallas import tpu_sc as plsc

BATCH = 16384
HIST_LEN = 50
EMBED_DIM = 32

SUB = 128
SG = BATCH // SUB
UNITS = HIST_LEN * SG
NC, NS = 2, 16
NW = NC * NS
UPW = UNITS // NW
DG = EMBED_DIM // 8

_MESH = plsc.VectorSubcoreMesh(core_axis_name="c", subcore_axis_name="s")

_TC_COLS = 4096
_TC_GRID = 62
_OUT_ROWS = _TC_GRID * _TC_COLS


def _tc_transpose_body(x0_ref, x1_ref, x2_ref, x3_ref, o_ref):
    parts = [x_ref[...].T for x_ref in (x0_ref, x1_ref, x2_ref, x3_ref)]
    o_ref[...] = jnp.concatenate(parts, axis=1)


def _tc_transpose(table_t):
    last_block = (1000000 - 1) // _TC_COLS
    specs = [
        pl.BlockSpec(
            (EMBED_DIM, _TC_COLS),
            functools.partial(
                lambda a, j: (0, jnp.minimum(4 * j + a, last_block)), a))
        for a in range(4)
    ]
    return pl.pallas_call(
        _tc_transpose_body,
        grid=(_TC_GRID,),
        in_specs=specs,
        out_specs=pl.BlockSpec((_TC_COLS, 128), lambda j: (j, 0)),
        out_shape=jax.ShapeDtypeStruct((_OUT_ROWS, 128), jnp.float32),
    )(table_t, table_t, table_t, table_t)


@functools.partial(
    pl.kernel,
    mesh=_MESH,
    out_type=jax.ShapeDtypeStruct((HIST_LEN, DG, SG, 8, SUB), jnp.float32),
    compiler_params=pltpu.CompilerParams(
        use_tc_tiling_on_sc=False, needs_layout_passes=False),
    scratch_types=[
        pltpu.VMEM((UPW, SUB), jnp.int32),
        pltpu.VMEM((4, SUB, EMBED_DIM), jnp.float32),
        pltpu.VMEM((2, EMBED_DIM, SUB), jnp.float32),
        pltpu.SemaphoreType.DMA((4,)),
        pltpu.SemaphoreType.DMA((2,)),
    ],
)
def _gather_sc(table_hbm, idx_hbm, out_hbm, idx_all, rows_g, rows_t, sem_g, sem_o):
    wid = lax.axis_index("s") * NC + lax.axis_index("c")
    u0 = wid * UPW

    pltpu.sync_copy(idx_hbm.at[pl.ds(u0, UPW)], idx_all)

    def gather(t):
        q = lax.rem(t, 4)
        return pltpu.make_async_copy(
            table_hbm.at[idx_all.at[t]], rows_g.at[q], sem_g.at[q])

    def store(t, dg):
        u = u0 + t
        h = lax.div(u, SG)
        sg = lax.rem(u, SG)
        q = lax.rem(t, 2)
        return pltpu.make_async_copy(
            rows_t.at[q, pl.ds(dg * 8, 8)], out_hbm.at[h, dg, sg], sem_o.at[q])

    lanevec = lax.iota(jnp.int32, 16)
    riota = [lanevec + 16 * k for k in range(8)]
    cvecs = [(lanevec + d0) & (EMBED_DIM - 1) for d0 in range(EMBED_DIM)]

    gather(0).start()
    gather(1).start()
    gather(2).start()

    def unit(t, carry):
        q4 = lax.rem(t, 4)
        q2 = lax.rem(t, 2)

        @pl.when(t >= 2)
        def _():
            for dg in range(DG):
                store(t - 2, dg).wait()

        gather(t).wait()

        @pl.when(t + 3 < UPW)
        def _():
            gather(t + 3).start()

        src = rows_g.at[q4]
        qvec = jnp.full((16,), 0, jnp.int32) + q2

        for d0 in range(0, EMBED_DIM, 2):
            c0, c1 = cvecs[d0], cvecs[d0 + 1]
            vs = [plsc.load_gather(src, [riota[k], c0]) for k in range(8)]
            vs += [plsc.load_gather(src, [riota[k], c1]) for k in range(8)]
            for k in range(8):
                plsc.store_scatter(rows_t, [qvec, c0, riota[k]], vs[k])
            for k in range(8):
                plsc.store_scatter(rows_t, [qvec, c1, riota[k]], vs[8 + k])

        for dg in range(DG):
            store(t, dg).start()
        return carry

    lax.fori_loop(0, UPW, unit, 0)

    for dg in range(DG):
        store(UPW - 2, dg).wait()
        store(UPW - 1, dg).wait()


def kernel(nouns_idx_tensor, histwords_embeddings):
    idx = nouns_idx_tensor.astype(jnp.int32).T.reshape(UNITS, SUB)
    idx = ((idx >> 14) << 14) + ((idx & 4095) << 2) + ((idx >> 12) & 3)
    table_l = _tc_transpose(histwords_embeddings.T)
    out5 = _gather_sc(table_l.reshape(4 * _OUT_ROWS, EMBED_DIM), idx)
    return out5.transpose(2, 4, 0, 1, 3).reshape(BATCH, HIST_LEN, EMBED_DIM)


# speedup vs baseline: 2.2890x; 1.0219x over previous
import functools

import jax
import jax.numpy as jnp
from jax import lax
from jax.experimental import pallas as pl
from jax.experimental.pallas import tpu as pltpu
from jax.experimental.pallas import tpu_sc as plsc

BATCH = 16384
HIST_LEN = 50
EMBED_DIM = 32

SUB = 128
SG = BATCH // SUB
UNITS = HIST_LEN * SG
NC, NS = 2, 16
NW = NC * NS
UPW = UNITS // NW
DG = EMBED_DIM // 8

_MESH = plsc.VectorSubcoreMesh(core_axis_name="c", subcore_axis_name="s")

_TC_COLS = 4096
_TC_GRID = 62
_OUT_ROWS = _TC_GRID * _TC_COLS


def _tc_transpose_body(x0_ref, x1_ref, x2_ref, x3_ref, o_ref):
    parts = [x_ref[...].T for x_ref in (x0_ref, x1_ref, x2_ref, x3_ref)]
    o_ref[...] = jnp.concatenate(parts, axis=1)


def _tc_transpose(table_t):
    last_block = (1000000 - 1) // _TC_COLS
    specs = [
        pl.BlockSpec(
            (EMBED_DIM, _TC_COLS),
            functools.partial(
                lambda a, j: (0, jnp.minimum(4 * j + a, last_block)), a))
        for a in range(4)
    ]
    return pl.pallas_call(
        _tc_transpose_body,
        grid=(_TC_GRID,),
        in_specs=specs,
        out_specs=pl.BlockSpec((_TC_COLS, 128), lambda j: (j, 0)),
        out_shape=jax.ShapeDtypeStruct((_OUT_ROWS, 128), jnp.float32),
    )(table_t, table_t, table_t, table_t)


@functools.partial(
    pl.kernel,
    mesh=_MESH,
    out_type=jax.ShapeDtypeStruct((HIST_LEN, DG, SG, 8, SUB), jnp.float32),
    compiler_params=pltpu.CompilerParams(
        use_tc_tiling_on_sc=False, needs_layout_passes=False),
    scratch_types=[
        pltpu.VMEM((UPW, SUB), jnp.int32),
        pltpu.VMEM((4, SUB, EMBED_DIM), jnp.float32),
        pltpu.VMEM((2, EMBED_DIM, SUB), jnp.float32),
        pltpu.SemaphoreType.DMA((4,)),
        pltpu.SemaphoreType.DMA((2,)),
    ],
)
def _gather_sc(table_hbm, idx_hbm, out_hbm, idx_all, rows_g, rows_t, sem_g, sem_o):
    wid = lax.axis_index("s") * NC + lax.axis_index("c")
    u0 = wid * UPW

    pltpu.sync_copy(idx_hbm.at[pl.ds(u0, UPW)], idx_all)

    def gather(t):
        q = lax.rem(t, 4)
        return pltpu.make_async_copy(
            table_hbm.at[idx_all.at[t]], rows_g.at[q], sem_g.at[q])

    def store(t, dg):
        u = u0 + t
        h = lax.div(u, SG)
        sg = lax.rem(u, SG)
        q = lax.rem(t, 2)
        return pltpu.make_async_copy(
            rows_t.at[q, pl.ds(dg * 8, 8)], out_hbm.at[h, dg, sg], sem_o.at[q])

    lanevec = lax.iota(jnp.int32, 16)
    riota = [lanevec + 16 * k for k in range(8)]
    cvecs = [(lanevec + d0) & (EMBED_DIM - 1) for d0 in range(EMBED_DIM)]

    gather(0).start()
    gather(1).start()
    gather(2).start()

    def unit(t, carry):
        q4 = lax.rem(t, 4)
        q2 = lax.rem(t, 2)

        @pl.when(t >= 2)
        def _():
            for dg in range(DG):
                store(t - 2, dg).wait()

        gather(t).wait()

        @pl.when(t + 3 < UPW)
        def _():
            gather(t + 3).start()

        src = rows_g.at[q4]
        qvec = jnp.full((16,), 0, jnp.int32) + q2

        for d0 in range(0, EMBED_DIM, 4):
            cs = [cvecs[d0 + i] for i in range(4)]
            vs = [plsc.load_gather(src, [riota[k], c])
                  for c in cs for k in range(8)]
            for i, c in enumerate(cs):
                for k in range(8):
                    plsc.store_scatter(rows_t, [qvec, c, riota[k]],
                                       vs[8 * i + k])

        for dg in range(DG):
            store(t, dg).start()
        return carry

    lax.fori_loop(0, UPW, unit, 0)

    for dg in range(DG):
        store(UPW - 2, dg).wait()
        store(UPW - 1, dg).wait()


def kernel(nouns_idx_tensor, histwords_embeddings):
    idx = nouns_idx_tensor.astype(jnp.int32).T.reshape(UNITS, SUB)
    idx = ((idx >> 14) << 14) + ((idx & 4095) << 2) + ((idx >> 12) & 3)
    table_l = _tc_transpose(histwords_embeddings.T)
    out5 = _gather_sc(table_l.reshape(4 * _OUT_ROWS, EMBED_DIM), idx)
    return out5.transpose(2, 4, 0, 1, 3).reshape(BATCH, HIST_LEN, EMBED_DIM)


# speedup vs baseline: 2.3217x; 1.0143x over previous
import functools

import jax
import jax.numpy as jnp
from jax import lax
from jax.experimental import pallas as pl
from jax.experimental.pallas import tpu as pltpu
from jax.experimental.pallas import tpu_sc as plsc

BATCH = 16384
HIST_LEN = 50
EMBED_DIM = 32

SUB = 128
SG = BATCH // SUB
UNITS = HIST_LEN * SG
NC, NS = 2, 16
NW = NC * NS
UPW = UNITS // NW
DG = EMBED_DIM // 8

_MESH = plsc.VectorSubcoreMesh(core_axis_name="c", subcore_axis_name="s")

_TC_COLS = 8192
_TC_GRID = 31
_OUT_ROWS = _TC_GRID * _TC_COLS


def _tc_transpose_body(x0_ref, x1_ref, x2_ref, x3_ref, o_ref):
    parts = [x_ref[...].T for x_ref in (x0_ref, x1_ref, x2_ref, x3_ref)]
    o_ref[...] = jnp.concatenate(parts, axis=1)


def _tc_transpose(table_t):
    last_block = (1000000 - 1) // _TC_COLS
    specs = [
        pl.BlockSpec(
            (EMBED_DIM, _TC_COLS),
            functools.partial(
                lambda a, j: (0, jnp.minimum(4 * j + a, last_block)), a))
        for a in range(4)
    ]
    return pl.pallas_call(
        _tc_transpose_body,
        grid=(_TC_GRID,),
        in_specs=specs,
        out_specs=pl.BlockSpec((_TC_COLS, 128), lambda j: (j, 0)),
        out_shape=jax.ShapeDtypeStruct((_OUT_ROWS, 128), jnp.float32),
    )(table_t, table_t, table_t, table_t)


@functools.partial(
    pl.kernel,
    mesh=_MESH,
    out_type=jax.ShapeDtypeStruct((HIST_LEN, DG, SG, 8, SUB), jnp.float32),
    compiler_params=pltpu.CompilerParams(
        use_tc_tiling_on_sc=False, needs_layout_passes=False),
    scratch_types=[
        pltpu.VMEM((UPW, SUB), jnp.int32),
        pltpu.VMEM((4, SUB, EMBED_DIM), jnp.float32),
        pltpu.VMEM((2, EMBED_DIM, SUB), jnp.float32),
        pltpu.SemaphoreType.DMA((4,)),
        pltpu.SemaphoreType.DMA((2,)),
    ],
)
def _gather_sc(table_hbm, idx_hbm, out_hbm, idx_all, rows_g, rows_t, sem_g, sem_o):
    wid = lax.axis_index("s") * NC + lax.axis_index("c")
    u0 = wid * UPW

    pltpu.sync_copy(idx_hbm.at[pl.ds(u0, UPW)], idx_all)

    def gather(t):
        q = lax.rem(t, 4)
        return pltpu.make_async_copy(
            table_hbm.at[idx_all.at[t]], rows_g.at[q], sem_g.at[q])

    def store(t, dg):
        u = u0 + t
        h = lax.div(u, SG)
        sg = lax.rem(u, SG)
        q = lax.rem(t, 2)
        return pltpu.make_async_copy(
            rows_t.at[q, pl.ds(dg * 8, 8)], out_hbm.at[h, dg, sg], sem_o.at[q])

    lanevec = lax.iota(jnp.int32, 16)
    riota = [lanevec + 16 * k for k in range(8)]
    cvecs = [(lanevec + d0) & (EMBED_DIM - 1) for d0 in range(EMBED_DIM)]

    gather(0).start()
    gather(1).start()
    gather(2).start()

    def unit(t, carry):
        q4 = lax.rem(t, 4)
        q2 = lax.rem(t, 2)

        @pl.when(t >= 2)
        def _():
            for dg in range(DG):
                store(t - 2, dg).wait()

        gather(t).wait()

        @pl.when(t + 3 < UPW)
        def _():
            gather(t + 3).start()

        src = rows_g.at[q4]
        qvec = jnp.full((16,), 0, jnp.int32) + q2

        for d0 in range(0, EMBED_DIM, 4):
            cs = [cvecs[d0 + i] for i in range(4)]
            vs = [plsc.load_gather(src, [riota[k], c])
                  for c in cs for k in range(8)]
            for i, c in enumerate(cs):
                for k in range(8):
                    plsc.store_scatter(rows_t, [qvec, c, riota[k]],
                                       vs[8 * i + k])

        for dg in range(DG):
            store(t, dg).start()
        return carry

    lax.fori_loop(0, UPW, unit, 0)

    for dg in range(DG):
        store(UPW - 2, dg).wait()
        store(UPW - 1, dg).wait()


def kernel(nouns_idx_tensor, histwords_embeddings):
    idx = nouns_idx_tensor.astype(jnp.int32).T.reshape(UNITS, SUB)
    idx = ((idx >> 15) << 15) + ((idx & 8191) << 2) + ((idx >> 13) & 3)
    table_l = _tc_transpose(histwords_embeddings.T)
    out5 = _gather_sc(table_l.reshape(4 * _OUT_ROWS, EMBED_DIM), idx)
    return out5.transpose(2, 4, 0, 1, 3).reshape(BATCH, HIST_LEN, EMBED_DIM)


# speedup vs baseline: 4.0572x; 1.7475x over previous
import functools

import jax
import jax.numpy as jnp
from jax import lax
from jax.experimental import pallas as pl
from jax.experimental.pallas import tpu as pltpu
from jax.experimental.pallas import tpu_sc as plsc

BATCH = 16384
HIST_LEN = 50
EMBED_DIM = 32

SUB = 128
SG = BATCH // SUB
UNITS = HIST_LEN * SG
NC, NS = 2, 16
NW = NC * NS
UPW = UNITS // NW
DG = EMBED_DIM // 8

_MESH = plsc.VectorSubcoreMesh(core_axis_name="c", subcore_axis_name="s")

_TC_COLS = 8192
_TC_GRID = 31
_OUT_ROWS = _TC_GRID * _TC_COLS


def _tc_transpose_body(x0_ref, x1_ref, x2_ref, x3_ref, o_ref):
    stack = jnp.concatenate(
        [x_ref[...] for x_ref in (x0_ref, x1_ref, x2_ref, x3_ref)], axis=0)
    o_ref[...] = stack.T


def _tc_transpose(table_t):
    last_block = (1000000 - 1) // _TC_COLS
    specs = [
        pl.BlockSpec(
            (EMBED_DIM, _TC_COLS),
            functools.partial(
                lambda a, j: (0, jnp.minimum(4 * j + a, last_block)), a))
        for a in range(4)
    ]
    return pl.pallas_call(
        _tc_transpose_body,
        grid=(_TC_GRID,),
        in_specs=specs,
        out_specs=pl.BlockSpec((_TC_COLS, 128), lambda j: (j, 0)),
        out_shape=jax.ShapeDtypeStruct((_OUT_ROWS, 128), jnp.float32),
    )(table_t, table_t, table_t, table_t)


@functools.partial(
    pl.kernel,
    mesh=_MESH,
    out_type=jax.ShapeDtypeStruct((HIST_LEN, DG, SG, 8, SUB), jnp.float32),
    compiler_params=pltpu.CompilerParams(
        use_tc_tiling_on_sc=False, needs_layout_passes=False),
    scratch_types=[
        pltpu.VMEM((UPW, SUB), jnp.int32),
        pltpu.VMEM((4, SUB, EMBED_DIM), jnp.float32),
        pltpu.VMEM((2, EMBED_DIM, SUB), jnp.float32),
        pltpu.SemaphoreType.DMA((4,)),
        pltpu.SemaphoreType.DMA((2,)),
    ],
)
def _gather_sc(table_hbm, idx_hbm, out_hbm, idx_all, rows_g, rows_t, sem_g, sem_o):
    wid = lax.axis_index("s") * NC + lax.axis_index("c")
    u0 = wid * UPW

    pltpu.sync_copy(idx_hbm.at[pl.ds(u0, UPW)], idx_all)

    def gather(t):
        q = lax.rem(t, 4)
        return pltpu.make_async_copy(
            table_hbm.at[idx_all.at[t]], rows_g.at[q], sem_g.at[q])

    def store(t, dg):
        u = u0 + t
        h = lax.div(u, SG)
        sg = lax.rem(u, SG)
        q = lax.rem(t, 2)
        return pltpu.make_async_copy(
            rows_t.at[q, pl.ds(dg * 8, 8)], out_hbm.at[h, dg, sg], sem_o.at[q])

    lanevec = lax.iota(jnp.int32, 16)
    riota = [lanevec + 16 * k for k in range(8)]
    cvecs = [(lanevec + d0) & (EMBED_DIM - 1) for d0 in range(EMBED_DIM)]

    gather(0).start()
    gather(1).start()
    gather(2).start()

    def unit(t, carry):
        q4 = lax.rem(t, 4)
        q2 = lax.rem(t, 2)

        @pl.when(t >= 2)
        def _():
            for dg in range(DG):
                store(t - 2, dg).wait()

        gather(t).wait()

        @pl.when(t + 3 < UPW)
        def _():
            gather(t + 3).start()

        src = rows_g.at[q4]
        qvec = jnp.full((16,), 0, jnp.int32) + q2

        for d0 in range(0, EMBED_DIM, 4):
            cs = [cvecs[d0 + i] for i in range(4)]
            vs = [plsc.load_gather(src, [riota[k], c])
                  for c in cs for k in range(8)]
            for i, c in enumerate(cs):
                for k in range(8):
                    plsc.store_scatter(rows_t, [qvec, c, riota[k]],
                                       vs[8 * i + k])

        for dg in range(DG):
            store(t, dg).start()
        return carry

    lax.fori_loop(0, UPW, unit, 0)

    for dg in range(DG):
        store(UPW - 2, dg).wait()
        store(UPW - 1, dg).wait()


def kernel(nouns_idx_tensor, histwords_embeddings):
    idx = nouns_idx_tensor.astype(jnp.int32).T.reshape(UNITS, SUB)
    idx = ((idx >> 15) << 15) + ((idx & 8191) << 2) + ((idx >> 13) & 3)
    table_l = _tc_transpose(histwords_embeddings.T)
    out5 = _gather_sc(table_l.reshape(4 * _OUT_ROWS, EMBED_DIM), idx)
    return out5.transpose(2, 4, 0, 1, 3).reshape(BATCH, HIST_LEN, EMBED_DIM)
